# baseline (speedup 1.0000x reference)
import functools

import jax
import jax.numpy as jnp
from jax import lax
from jax.experimental import pallas as pl
from jax.experimental.pallas import tpu as pltpu
from jax.experimental.pallas import tpu_sc as plsc

N = 10000
NPAD = 10240
NC, NS = 2, 16
NW = NC * NS
K = 128
ETOT = 320000 + N
STEPS = 84
EPAD = STEPS * NW * K
RPT = NPAD // NS

_mesh = plsc.VectorSubcoreMesh(
    core_axis_name="c", subcore_axis_name="s", num_cores=NC, num_subcores=NS)


def _nbd(D):
    return 2 if D == 128 else 6


@functools.partial(jax.jit, static_argnums=(2,))
def _sc_agg(g, edges, D):
    NB = _nbd(D)
    SS = STEPS // NB
    P = SS // 2

    @functools.partial(
        pl.kernel,
        out_type=jax.ShapeDtypeStruct((NC, NPAD, D), jnp.float32),
        mesh=_mesh,
        scratch_types=[
            pltpu.VMEM_SHARED((NPAD, D), jnp.float32),
            pltpu.VMEM((2 * NB, K), jnp.int32),
            pltpu.VMEM((2 * NB, K), jnp.int32),
        ] + [pltpu.VMEM((K, D), jnp.float32) for _ in range(NB)]
          + [pltpu.SemaphoreType.DMA for _ in range(2 * NB + 2)],
        compiler_params=pltpu.CompilerParams(use_tc_tiling_on_sc=False),
    )
    def agg(g_hbm, e_hbm, out_hbm, acc, eb0, eb1, *bufs):
        rows = bufs[:NB]
        gsem = bufs[NB:2 * NB]
        ssem = bufs[2 * NB:3 * NB]
        isem = bufs[3 * NB:]
        eb = (eb0, eb1)
        c = lax.axis_index("c")
        s = lax.axis_index("s")
        wid = s * NC + c
        ebase = wid * SS * 2 * NB

        def pf_idx(ss, pb):
            pltpu.async_copy(e_hbm.at[pl.ds(ebase + ss * 2 * NB, 2 * NB)],
                             eb[pb], isem[pb])

        def drain_idx(pb):
            pltpu.make_async_copy(e_hbm.at[pl.ds(0, 2 * NB)],
                                  eb[pb], isem[pb]).wait()

        def drain(sem, b):
            pltpu.make_async_copy(g_hbm.at[eb0.at[0]], rows[b], sem[b]).wait()

        def half(ss, pb, last):
            if not last:
                pf_idx(ss + 1, 1 - pb)
            for b in range(NB):
                drain(gsem, b)
                pltpu.async_copy(rows[b], acc.at[eb[pb].at[NB + b]], ssem[b],
                                 add=True)
            for b in range(NB):
                drain(ssem, b)
                if not last:
                    if b == 0:
                        drain_idx(1 - pb)
                    pltpu.async_copy(g_hbm.at[eb[1 - pb].at[b]], rows[b],
                                     gsem[b])

        for t in range(RPT // K):
            pltpu.sync_copy(g_hbm.at[pl.ds(NPAD - K, K)],
                            acc.at[pl.ds(s * RPT + t * K, K)])
        pf_idx(0, 0)
        plsc.subcore_barrier()
        drain_idx(0)
        for b in range(NB):
            pltpu.async_copy(g_hbm.at[eb0.at[b]], rows[b], gsem[b])

        def body(p, carry):
            half(2 * p, 0, False)
            half(2 * p + 1, 1, False)
            return carry

        lax.fori_loop(0, P - 1, body, 0)
        half(2 * P - 2, 0, False)
        half(2 * P - 1, 1, True)
        plsc.subcore_barrier()
        pltpu.sync_copy(acc.at[pl.ds(s * RPT, RPT)],
                        out_hbm.at[c, pl.ds(s * RPT, RPT)])

    return agg(g, edges)


B = 1024
G = NPAD // B


def _rspec(D):
    return pl.BlockSpec((B, D), lambda i: (i, 0))


def _pspec(D):
    return pl.BlockSpec((NC, B, D), lambda i: (0, i, 0))


def _wspec(shape):
    return pl.BlockSpec(shape, lambda i: tuple(0 for _ in shape))


def _tc_call(body, in_specs, ins, out_specs, out_shapes):
    return pl.pallas_call(
        body,
        grid=(G,),
        in_specs=in_specs,
        out_specs=out_specs,
        out_shape=[jax.ShapeDtypeStruct(s, jnp.float32) for s in out_shapes],
    )(*ins)


@jax.jit
def _tc0(degp, x, W1):
    def body(degp_ref, x_ref, w_ref, dis_ref, g_ref):
        deg = degp_ref[0, :, 0:1] + degp_ref[1, :, 0:1]
        dis = jnp.where(deg > 0, lax.rsqrt(jnp.maximum(deg, 1e-12)), 0.0)
        dis_ref[...] = dis
        g_ref[...] = dis * jnp.dot(x_ref[...], w_ref[...],
                                   preferred_element_type=jnp.float32)
    return _tc_call(
        body,
        [_pspec(16), _rspec(128), _wspec((128, 64))],
        (degp, x, W1),
        [_rspec(1), _rspec(64)],
        [(NPAD, 1), (NPAD, 64)])


@jax.jit
def _tc1(p, dis, b1):
    def body(p_ref, dis_ref, b_ref, g_ref):
        dis = dis_ref[...]
        a = dis * (p_ref[0] + p_ref[1]) + b_ref[...]
        g_ref[...] = dis * jnp.maximum(a, 0.0)
    return _tc_call(
        body,
        [_pspec(64), _rspec(1), _wspec((1, 64))],
        (p, dis, b1.reshape(1, 64)),
        [_rspec(64)],
        [(NPAD, 64)])[0]


@jax.jit
def _tc2(p, dis, W2, b2):
    def body(p_ref, dis_ref, w_ref, b_ref, ga_ref, gb_ref):
        dis = dis_ref[...]
        a = dis * (p_ref[0] + p_ref[1])
        t = jnp.dot(a, w_ref[...], preferred_element_type=jnp.float32)
        g = dis * jnp.maximum(t + b_ref[...], 0.0)
        ga_ref[...] = g[:, :128]
        gb_ref[...] = g[:, 128:]
    return _tc_call(
        body,
        [_pspec(64), _rspec(1), _wspec((64, 256)), _wspec((1, 256))],
        (p, dis, W2, b2.reshape(1, 256)),
        [_rspec(128), _rspec(128)],
        [(NPAD, 128), (NPAD, 128)])


@jax.jit
def _tc3(pa, pb, dis, W3, b3, W4):
    def body(pa_ref, pb_ref, dis_ref, wa_ref, wb_ref, b_ref, w4_ref,
             ga_ref, gb_ref):
        dis = dis_ref[...]
        t = (jnp.dot(dis * (pa_ref[0] + pa_ref[1]), wa_ref[...],
                     preferred_element_type=jnp.float32)
             + jnp.dot(dis * (pb_ref[0] + pb_ref[1]), wb_ref[...],
                       preferred_element_type=jnp.float32))
        o = jnp.maximum(t + b_ref[...], 0.0)
        g = dis * jnp.dot(o, w4_ref[...], preferred_element_type=jnp.float32)
        ga_ref[...] = g[:, :128]
        gb_ref[...] = g[:, 128:]
    return _tc_call(
        body,
        [_pspec(128), _pspec(128), _rspec(1), _wspec((128, 512)),
         _wspec((128, 512)), _wspec((1, 512)), _wspec((512, 256))],
        (pa, pb, dis, W3[:128], W3[128:], b3.reshape(1, 512), W4),
        [_rspec(128), _rspec(128)],
        [(NPAD, 128), (NPAD, 128)])


@jax.jit
def _tc4(pa, pb, dis, b4, W5):
    def body(pa_ref, pb_ref, dis_ref, ba_ref, bb_ref, wa_ref, wb_ref, g_ref):
        dis = dis_ref[...]
        ha = jnp.maximum(dis * (pa_ref[0] + pa_ref[1]) + ba_ref[...], 0.0)
        hb = jnp.maximum(dis * (pb_ref[0] + pb_ref[1]) + bb_ref[...], 0.0)
        g_ref[...] = dis * (
            jnp.dot(ha, wa_ref[...], preferred_element_type=jnp.float32)
            + jnp.dot(hb, wb_ref[...], preferred_element_type=jnp.float32))
    return _tc_call(
        body,
        [_pspec(128), _pspec(128), _rspec(1), _wspec((1, 128)),
         _wspec((1, 128)), _wspec((128, 64)), _wspec((128, 64))],
        (pa, pb, dis, b4[:128].reshape(1, 128), b4[128:].reshape(1, 128),
         W5[:128], W5[128:]),
        [_rspec(64)],
        [(NPAD, 64)])[0]


@jax.jit
def _tc5(p, dis, b5, W6):
    def body(p_ref, dis_ref, b_ref, w_ref, g_ref):
        dis = dis_ref[...]
        h = jnp.maximum(dis * (p_ref[0] + p_ref[1]) + b_ref[...], 0.0)
        g_ref[...] = dis * jnp.dot(h, w_ref[...],
                                   preferred_element_type=jnp.float32)
    return _tc_call(
        body,
        [_pspec(64), _rspec(1), _wspec((1, 64)), _wspec((64, 16))],
        (p, dis, b5.reshape(1, 64), W6),
        [_rspec(16)],
        [(NPAD, 16)])[0]


@jax.jit
def _tc6(p, dis, b6):
    def body(p_ref, dis_ref, b_ref, y_ref):
        y_ref[...] = dis_ref[...] * (p_ref[0] + p_ref[1]) + b_ref[...]
    return _tc_call(
        body,
        [_pspec(16), _rspec(1), _wspec((1, 16))],
        (p, dis, b6.reshape(1, 16)),
        [_rspec(16)],
        [(NPAD, 16)])[0]


def kernel(x, edge_index, W1, b1, W2, b2, W3, b3, W4, b4, W5, b5, W6, b6):
    loop = jnp.arange(N, dtype=jnp.int32)
    pad = jnp.full((EPAD - ETOT,), N, jnp.int32)
    src = jnp.concatenate([edge_index[0], loop, pad])
    dst = jnp.concatenate([edge_index[1], loop, pad])

    def pack(nb):
        ss = STEPS // nb
        s4 = src.reshape(NW, ss, nb, K)
        d4 = dst.reshape(NW, ss, nb, K)
        return jnp.concatenate([s4, d4], axis=2).reshape(-1, K)

    e2, e6 = pack(2), pack(6)
    xp = jnp.zeros((NPAD, 128), jnp.float32).at[:N].set(x)

    ones = jnp.zeros((NPAD, 16), jnp.float32).at[:N].set(1.0)
    degp = _sc_agg(ones, e6, 16)
    dis, g1 = _tc0(degp, xp, W1)
    p1 = _sc_agg(g1, e6, 64)
    g2 = _tc1(p1, dis, b1)
    p2 = _sc_agg(g2, e6, 64)
    g3a, g3b = _tc2(p2, dis, W2, b2)
    p3a = _sc_agg(g3a, e2, 128)
    p3b = _sc_agg(g3b, e2, 128)
    g4a, g4b = _tc3(p3a, p3b, dis, W3, b3, W4)
    p4a = _sc_agg(g4a, e2, 128)
    p4b = _sc_agg(g4b, e2, 128)
    g5 = _tc4(p4a, p4b, dis, b4, W5)
    p5 = _sc_agg(g5, e6, 64)
    g6 = _tc5(p5, dis, b5, W6)
    p6 = _sc_agg(g6, e6, 16)
    y = _tc6(p6, dis, b6)
    return y[:N]


# speedup vs baseline: 2.7685x; 2.7685x over previous
import functools

import jax
import jax.numpy as jnp
from jax import lax
from jax.experimental import pallas as pl
from jax.experimental.pallas import tpu as pltpu
from jax.experimental.pallas import tpu_sc as plsc

N = 10000
NPAD = 10240
NC, NS = 2, 16
NW = NC * NS
K = 128
ETOT = 320000 + N
STEPS = 84
EPAD = STEPS * NW * K
RPT = NPAD // NS

_mesh = plsc.VectorSubcoreMesh(
    core_axis_name="c", subcore_axis_name="s", num_cores=NC, num_subcores=NS)


NB = 3
SS = STEPS // NB


@functools.partial(jax.jit, static_argnums=(3,))
def _sc_agg(g, src2, dst2, D):
    DS = min(D, 64)
    SLABS = D // DS

    @functools.partial(
        pl.kernel,
        out_type=jax.ShapeDtypeStruct((NC, NPAD, D), jnp.float32),
        mesh=_mesh,
        scratch_types=[
            pltpu.VMEM_SHARED((NPAD, DS), jnp.float32),
            pltpu.VMEM_SHARED((NPAD, DS), jnp.float32),
            pltpu.VMEM((STEPS, K), jnp.int32),
            pltpu.VMEM((STEPS, K), jnp.int32),
        ] + [pltpu.VMEM((K, DS), jnp.float32) for _ in range(NB)]
          + [pltpu.SemaphoreType.DMA for _ in range(2 * NB)],
        compiler_params=pltpu.CompilerParams(use_tc_tiling_on_sc=False),
    )
    def agg(g_hbm, src_hbm, dst_hbm, out_hbm, g_sp, acc, sbuf, dbuf, *bufs):
        rows = bufs[:NB]
        gsem = bufs[NB:2 * NB]
        ssem = bufs[2 * NB:]
        c = lax.axis_index("c")
        s = lax.axis_index("s")
        wid = s * NC + c
        pltpu.sync_copy(src_hbm.at[pl.ds(wid * STEPS, STEPS)], sbuf)
        pltpu.sync_copy(dst_hbm.at[pl.ds(wid * STEPS, STEPS)], dbuf)

        def drain(sem, b):
            pltpu.make_async_copy(
                g_hbm.at[pl.ds(0, K), pl.ds(0, DS)] if SLABS > 1
                else g_hbm.at[pl.ds(0, K)],
                rows[b], sem[b]).wait()

        for j in range(SLABS):
            r0 = s * RPT
            if SLABS > 1:
                pltpu.sync_copy(g_hbm.at[pl.ds(r0, RPT), pl.ds(j * DS, DS)],
                                g_sp.at[pl.ds(r0, RPT)])
                for t in range(RPT // K):
                    pltpu.sync_copy(
                        g_hbm.at[pl.ds(NPAD - K, K), pl.ds(j * DS, DS)],
                        acc.at[pl.ds(r0 + t * K, K)])
            else:
                pltpu.sync_copy(g_hbm.at[pl.ds(r0, RPT)],
                                g_sp.at[pl.ds(r0, RPT)])
                for t in range(RPT // K):
                    pltpu.sync_copy(g_hbm.at[pl.ds(NPAD - K, K)],
                                    acc.at[pl.ds(r0 + t * K, K)])
            plsc.subcore_barrier()

            for b in range(NB):
                pltpu.async_copy(g_sp.at[sbuf.at[b]], rows[b], gsem[b])

            def body(ss, carry):
                t0 = ss * NB
                for b in range(NB):
                    drain(gsem, b)
                    pltpu.async_copy(rows[b], acc.at[dbuf.at[t0 + b]],
                                     ssem[b], add=True)
                for b in range(NB):
                    drain(ssem, b)
                    pltpu.async_copy(g_sp.at[sbuf.at[t0 + NB + b]], rows[b],
                                     gsem[b])
                return carry

            lax.fori_loop(0, SS - 1, body, 0)
            t0 = (SS - 1) * NB
            for b in range(NB):
                drain(gsem, b)
                pltpu.async_copy(rows[b], acc.at[dbuf.at[t0 + b]], ssem[b],
                                 add=True)
            for b in range(NB):
                drain(ssem, b)
            plsc.subcore_barrier()
            if SLABS > 1:
                pltpu.sync_copy(acc.at[pl.ds(r0, RPT)],
                                out_hbm.at[c, pl.ds(r0, RPT),
                                           pl.ds(j * DS, DS)])
            else:
                pltpu.sync_copy(acc.at[pl.ds(r0, RPT)],
                                out_hbm.at[c, pl.ds(r0, RPT)])

    return agg(g, src2, dst2)


B = 1024
G = NPAD // B


def _rspec(D):
    return pl.BlockSpec((B, D), lambda i: (i, 0))


def _pspec(D):
    return pl.BlockSpec((NC, B, D), lambda i: (0, i, 0))


def _wspec(shape):
    return pl.BlockSpec(shape, lambda i: tuple(0 for _ in shape))


def _tc_call(body, in_specs, ins, out_specs, out_shapes):
    return pl.pallas_call(
        body,
        grid=(G,),
        in_specs=in_specs,
        out_specs=out_specs,
        out_shape=[jax.ShapeDtypeStruct(s, jnp.float32) for s in out_shapes],
    )(*ins)


@jax.jit
def _tc0(degp, x, W1):
    def body(degp_ref, x_ref, w_ref, dis_ref, g_ref):
        deg = degp_ref[0, :, 0:1] + degp_ref[1, :, 0:1]
        dis = jnp.where(deg > 0, lax.rsqrt(jnp.maximum(deg, 1e-12)), 0.0)
        dis_ref[...] = dis
        g_ref[...] = dis * jnp.dot(x_ref[...], w_ref[...],
                                   preferred_element_type=jnp.float32)
    return _tc_call(
        body,
        [_pspec(16), _rspec(128), _wspec((128, 64))],
        (degp, x, W1),
        [_rspec(1), _rspec(64)],
        [(NPAD, 1), (NPAD, 64)])


@jax.jit
def _tc1(p, dis, b1):
    def body(p_ref, dis_ref, b_ref, g_ref):
        dis = dis_ref[...]
        a = dis * (p_ref[0] + p_ref[1]) + b_ref[...]
        g_ref[...] = dis * jnp.maximum(a, 0.0)
    return _tc_call(
        body,
        [_pspec(64), _rspec(1), _wspec((1, 64))],
        (p, dis, b1.reshape(1, 64)),
        [_rspec(64)],
        [(NPAD, 64)])[0]


@jax.jit
def _tc2(p, dis, W2, b2):
    def body(p_ref, dis_ref, w_ref, b_ref, g_ref):
        dis = dis_ref[...]
        a = dis * (p_ref[0] + p_ref[1])
        t = jnp.dot(a, w_ref[...], preferred_element_type=jnp.float32)
        g_ref[...] = dis * jnp.maximum(t + b_ref[...], 0.0)
    return _tc_call(
        body,
        [_pspec(64), _rspec(1), _wspec((64, 256)), _wspec((1, 256))],
        (p, dis, W2, b2.reshape(1, 256)),
        [_rspec(256)],
        [(NPAD, 256)])[0]


@jax.jit
def _tc3(p, dis, W3, b3, W4):
    def body(p_ref, dis_ref, w3_ref, b_ref, w4_ref, g_ref):
        dis = dis_ref[...]
        t = jnp.dot(dis * (p_ref[0] + p_ref[1]), w3_ref[...],
                    preferred_element_type=jnp.float32)
        o = jnp.maximum(t + b_ref[...], 0.0)
        g_ref[...] = dis * jnp.dot(o, w4_ref[...],
                                   preferred_element_type=jnp.float32)
    return _tc_call(
        body,
        [_pspec(256), _rspec(1), _wspec((256, 512)), _wspec((1, 512)),
         _wspec((512, 256))],
        (p, dis, W3, b3.reshape(1, 512), W4),
        [_rspec(256)],
        [(NPAD, 256)])[0]


@jax.jit
def _tc4(p, dis, b4, W5):
    def body(p_ref, dis_ref, b_ref, w_ref, g_ref):
        dis = dis_ref[...]
        h = jnp.maximum(dis * (p_ref[0] + p_ref[1]) + b_ref[...], 0.0)
        g_ref[...] = dis * jnp.dot(h, w_ref[...],
                                   preferred_element_type=jnp.float32)
    return _tc_call(
        body,
        [_pspec(256), _rspec(1), _wspec((1, 256)), _wspec((256, 64))],
        (p, dis, b4.reshape(1, 256), W5),
        [_rspec(64)],
        [(NPAD, 64)])[0]


@jax.jit
def _tc5(p, dis, b5, W6):
    def body(p_ref, dis_ref, b_ref, w_ref, g_ref):
        dis = dis_ref[...]
        h = jnp.maximum(dis * (p_ref[0] + p_ref[1]) + b_ref[...], 0.0)
        g_ref[...] = dis * jnp.dot(h, w_ref[...],
                                   preferred_element_type=jnp.float32)
    return _tc_call(
        body,
        [_pspec(64), _rspec(1), _wspec((1, 64)), _wspec((64, 16))],
        (p, dis, b5.reshape(1, 64), W6),
        [_rspec(16)],
        [(NPAD, 16)])[0]


@jax.jit
def _tc6(p, dis, b6):
    def body(p_ref, dis_ref, b_ref, y_ref):
        y_ref[...] = dis_ref[...] * (p_ref[0] + p_ref[1]) + b_ref[...]
    return _tc_call(
        body,
        [_pspec(16), _rspec(1), _wspec((1, 16))],
        (p, dis, b6.reshape(1, 16)),
        [_rspec(16)],
        [(NPAD, 16)])[0]


def kernel(x, edge_index, W1, b1, W2, b2, W3, b3, W4, b4, W5, b5, W6, b6):
    loop = jnp.arange(N, dtype=jnp.int32)
    pad = jnp.full((EPAD - ETOT,), N, jnp.int32)
    src = jnp.concatenate([edge_index[0], loop, pad]).reshape(-1, K)
    dst = jnp.concatenate([edge_index[1], loop, pad]).reshape(-1, K)
    xp = jnp.zeros((NPAD, 128), jnp.float32).at[:N].set(x)

    ones = jnp.zeros((NPAD, 16), jnp.float32).at[:N].set(1.0)
    degp = _sc_agg(ones, src, dst, 16)
    dis, g1 = _tc0(degp, xp, W1)
    p1 = _sc_agg(g1, src, dst, 64)
    g2 = _tc1(p1, dis, b1)
    p2 = _sc_agg(g2, src, dst, 64)
    g3 = _tc2(p2, dis, W2, b2)
    p3 = _sc_agg(g3, src, dst, 256)
    g4 = _tc3(p3, dis, W3, b3, W4)
    p4 = _sc_agg(g4, src, dst, 256)
    g5 = _tc4(p4, dis, b4, W5)
    p5 = _sc_agg(g5, src, dst, 64)
    g6 = _tc5(p5, dis, b5, W6)
    p6 = _sc_agg(g6, src, dst, 16)
    y = _tc6(p6, dis, b6)
    return y[:N]


# speedup vs baseline: 2.8871x; 1.0428x over previous
import functools

import jax
import jax.numpy as jnp
from jax import lax
from jax.experimental import pallas as pl
from jax.experimental.pallas import tpu as pltpu
from jax.experimental.pallas import tpu_sc as plsc

N = 10000
NPAD = 10240
NC, NS = 2, 16
NW = NC * NS
K = 128
ETOT = 320000 + N
STEPS = 81
EPAD = STEPS * NW * K
RPT = NPAD // NS

_mesh = plsc.VectorSubcoreMesh(
    core_axis_name="c", subcore_axis_name="s", num_cores=NC, num_subcores=NS)


NB = 3
SS = STEPS // NB


@functools.partial(jax.jit, static_argnums=(3,))
def _sc_agg(g, src2, dst2, D):
    DS = min(D, 64)
    SLABS = D // DS

    @functools.partial(
        pl.kernel,
        out_type=jax.ShapeDtypeStruct((NC, NPAD, D), jnp.float32),
        mesh=_mesh,
        scratch_types=[
            pltpu.VMEM_SHARED((NPAD, DS), jnp.float32),
            pltpu.VMEM_SHARED((NPAD, DS), jnp.float32),
            pltpu.VMEM((STEPS, K), jnp.int32),
            pltpu.VMEM((STEPS, K), jnp.int32),
        ] + [pltpu.VMEM((K, DS), jnp.float32) for _ in range(NB)]
          + [pltpu.SemaphoreType.DMA for _ in range(2 * NB)],
        compiler_params=pltpu.CompilerParams(use_tc_tiling_on_sc=False),
    )
    def agg(g_hbm, src_hbm, dst_hbm, out_hbm, g_sp, acc, sbuf, dbuf, *bufs):
        rows = bufs[:NB]
        gsem = bufs[NB:2 * NB]
        ssem = bufs[2 * NB:]
        c = lax.axis_index("c")
        s = lax.axis_index("s")
        wid = s * NC + c
        pltpu.sync_copy(src_hbm.at[pl.ds(wid * STEPS, STEPS)], sbuf)
        pltpu.sync_copy(dst_hbm.at[pl.ds(wid * STEPS, STEPS)], dbuf)

        def drain(sem, b):
            pltpu.make_async_copy(
                g_hbm.at[pl.ds(0, K), pl.ds(0, DS)] if SLABS > 1
                else g_hbm.at[pl.ds(0, K)],
                rows[b], sem[b]).wait()

        for j in range(SLABS):
            r0 = s * RPT
            if SLABS > 1:
                pltpu.sync_copy(g_hbm.at[pl.ds(r0, RPT), pl.ds(j * DS, DS)],
                                g_sp.at[pl.ds(r0, RPT)])
                for t in range(RPT // K):
                    pltpu.sync_copy(
                        g_hbm.at[pl.ds(NPAD - K, K), pl.ds(j * DS, DS)],
                        acc.at[pl.ds(r0 + t * K, K)])
            else:
                pltpu.sync_copy(g_hbm.at[pl.ds(r0, RPT)],
                                g_sp.at[pl.ds(r0, RPT)])
                for t in range(RPT // K):
                    pltpu.sync_copy(g_hbm.at[pl.ds(NPAD - K, K)],
                                    acc.at[pl.ds(r0 + t * K, K)])
            plsc.subcore_barrier()

            for b in range(NB):
                pltpu.async_copy(g_sp.at[sbuf.at[b]], rows[b], gsem[b])

            def body(ss, carry):
                t0 = ss * NB
                for b in range(NB):
                    drain(gsem, b)
                    pltpu.async_copy(rows[b], acc.at[dbuf.at[t0 + b]],
                                     ssem[b], add=True)
                for b in range(NB):
                    drain(ssem, b)
                    pltpu.async_copy(g_sp.at[sbuf.at[t0 + NB + b]], rows[b],
                                     gsem[b])
                return carry

            lax.fori_loop(0, SS - 1, body, 0)
            t0 = (SS - 1) * NB
            for b in range(NB):
                drain(gsem, b)
                pltpu.async_copy(rows[b], acc.at[dbuf.at[t0 + b]], ssem[b],
                                 add=True)
            for b in range(NB):
                drain(ssem, b)
            plsc.subcore_barrier()
            if SLABS > 1:
                pltpu.sync_copy(acc.at[pl.ds(r0, RPT)],
                                out_hbm.at[c, pl.ds(r0, RPT),
                                           pl.ds(j * DS, DS)])
            else:
                pltpu.sync_copy(acc.at[pl.ds(r0, RPT)],
                                out_hbm.at[c, pl.ds(r0, RPT)])

    return agg(g, src2, dst2)


B = 1024
G = NPAD // B


def _rspec(D):
    return pl.BlockSpec((B, D), lambda i: (i, 0))


def _pspec(D):
    return pl.BlockSpec((NC, B, D), lambda i: (0, i, 0))


def _wspec(shape):
    return pl.BlockSpec(shape, lambda i: tuple(0 for _ in shape))


def _tc_call(body, in_specs, ins, out_specs, out_shapes):
    return pl.pallas_call(
        body,
        grid=(G,),
        in_specs=in_specs,
        out_specs=out_specs,
        out_shape=[jax.ShapeDtypeStruct(s, jnp.float32) for s in out_shapes],
    )(*ins)


@jax.jit
def _tc0(degp, x, W1):
    def body(degp_ref, x_ref, w_ref, dis_ref, g_ref):
        deg = degp_ref[0, :, 0:1] + degp_ref[1, :, 0:1]
        dis = jnp.where(deg > 0, lax.rsqrt(jnp.maximum(deg, 1e-12)), 0.0)
        dis_ref[...] = dis
        g_ref[...] = dis * jnp.dot(x_ref[...], w_ref[...],
                                   preferred_element_type=jnp.float32)
    return _tc_call(
        body,
        [_pspec(16), _rspec(128), _wspec((128, 64))],
        (degp, x, W1),
        [_rspec(1), _rspec(64)],
        [(NPAD, 1), (NPAD, 64)])


@jax.jit
def _tc1(p, dis, b1):
    def body(p_ref, dis_ref, b_ref, g_ref):
        dis = dis_ref[...]
        a = dis * (p_ref[0] + p_ref[1]) + b_ref[...]
        g_ref[...] = dis * jnp.maximum(a, 0.0)
    return _tc_call(
        body,
        [_pspec(64), _rspec(1), _wspec((1, 64))],
        (p, dis, b1.reshape(1, 64)),
        [_rspec(64)],
        [(NPAD, 64)])[0]


@jax.jit
def _tc2(p, dis, W2, b2):
    def body(p_ref, dis_ref, w_ref, b_ref, g_ref):
        dis = dis_ref[...]
        a = dis * (p_ref[0] + p_ref[1])
        t = jnp.dot(a, w_ref[...], preferred_element_type=jnp.float32)
        g_ref[...] = dis * jnp.maximum(t + b_ref[...], 0.0)
    return _tc_call(
        body,
        [_pspec(64), _rspec(1), _wspec((64, 256)), _wspec((1, 256))],
        (p, dis, W2, b2.reshape(1, 256)),
        [_rspec(256)],
        [(NPAD, 256)])[0]


@jax.jit
def _tc3(p, dis, W3, b3, W4):
    def body(p_ref, dis_ref, w3_ref, b_ref, w4_ref, g_ref):
        dis = dis_ref[...]
        t = jnp.dot(dis * (p_ref[0] + p_ref[1]), w3_ref[...],
                    preferred_element_type=jnp.float32)
        o = jnp.maximum(t + b_ref[...], 0.0)
        g_ref[...] = dis * jnp.dot(o, w4_ref[...],
                                   preferred_element_type=jnp.float32)
    return _tc_call(
        body,
        [_pspec(256), _rspec(1), _wspec((256, 512)), _wspec((1, 512)),
         _wspec((512, 256))],
        (p, dis, W3, b3.reshape(1, 512), W4),
        [_rspec(256)],
        [(NPAD, 256)])[0]


@jax.jit
def _tc4(p, dis, b4, W5):
    def body(p_ref, dis_ref, b_ref, w_ref, g_ref):
        dis = dis_ref[...]
        h = jnp.maximum(dis * (p_ref[0] + p_ref[1]) + b_ref[...], 0.0)
        g_ref[...] = dis * jnp.dot(h, w_ref[...],
                                   preferred_element_type=jnp.float32)
    return _tc_call(
        body,
        [_pspec(256), _rspec(1), _wspec((1, 256)), _wspec((256, 64))],
        (p, dis, b4.reshape(1, 256), W5),
        [_rspec(64)],
        [(NPAD, 64)])[0]


@jax.jit
def _tc5(p, dis, b5, W6):
    def body(p_ref, dis_ref, b_ref, w_ref, g_ref):
        dis = dis_ref[...]
        h = jnp.maximum(dis * (p_ref[0] + p_ref[1]) + b_ref[...], 0.0)
        g_ref[...] = dis * jnp.dot(h, w_ref[...],
                                   preferred_element_type=jnp.float32)
    return _tc_call(
        body,
        [_pspec(64), _rspec(1), _wspec((1, 64)), _wspec((64, 16))],
        (p, dis, b5.reshape(1, 64), W6),
        [_rspec(16)],
        [(NPAD, 16)])[0]


@jax.jit
def _tc6(p, dis, b6):
    def body(p_ref, dis_ref, b_ref, y_ref):
        y_ref[...] = dis_ref[...] * (p_ref[0] + p_ref[1]) + b_ref[...]
    return _tc_call(
        body,
        [_pspec(16), _rspec(1), _wspec((1, 16))],
        (p, dis, b6.reshape(1, 16)),
        [_rspec(16)],
        [(NPAD, 16)])[0]


def kernel(x, edge_index, W1, b1, W2, b2, W3, b3, W4, b4, W5, b5, W6, b6):
    loop = jnp.arange(N, dtype=jnp.int32)
    pad = jnp.full((EPAD - ETOT,), N, jnp.int32)
    src = jnp.concatenate([edge_index[0], loop, pad]).reshape(-1, K)
    dst = jnp.concatenate([edge_index[1], loop, pad]).reshape(-1, K)
    xp = jnp.zeros((NPAD, 128), jnp.float32).at[:N].set(x)

    ones = jnp.zeros((NPAD, 16), jnp.float32).at[:N].set(1.0)
    degp = _sc_agg(ones, src, dst, 16)
    dis, g1 = _tc0(degp, xp, W1)
    p1 = _sc_agg(g1, src, dst, 64)
    g2 = _tc1(p1, dis, b1)
    p2 = _sc_agg(g2, src, dst, 64)
    g3 = _tc2(p2, dis, W2, b2)
    p3 = _sc_agg(g3, src, dst, 256)
    g4 = _tc3(p3, dis, W3, b3, W4)
    p4 = _sc_agg(g4, src, dst, 256)
    g5 = _tc4(p4, dis, b4, W5)
    p5 = _sc_agg(g5, src, dst, 64)
    g6 = _tc5(p5, dis, b5, W6)
    p6 = _sc_agg(g6, src, dst, 16)
    y = _tc6(p6, dis, b6)
    return y[:N]
